# baseline (speedup 1.0000x reference)
import functools

import jax
import jax.numpy as jnp
from jax import lax
from jax.experimental import pallas as pl
from jax.experimental.pallas import tpu as pltpu
from jax.experimental.pallas import tpu_sc as plsc

_N = 10000
_E = 320000
_D = 128
_G = 64
_C = 16

_NC = 2
_NS = 16
_K = 128

_NPAD = 10112
_RPS = _NPAD // _NS

_EP = _E + _N
_NPH = 2
_NITER = 42
_EPAD = _NC * _NS * _NPH * _NITER * _K

_mesh = plsc.VectorSubcoreMesh(core_axis_name="c", subcore_axis_name="s")


@functools.partial(
    pl.kernel,
    out_type=jax.ShapeDtypeStruct((_NC, _NPAD, _D), jnp.float32),
    mesh=_mesh,
    scratch_types=[
        pltpu.VMEM((_NITER, _K), jnp.int32),
        pltpu.VMEM((_K, _D), jnp.float32),
        pltpu.VMEM_SHARED((_NPAD, _D), jnp.float32),
    ],
)
def _deg_kernel(dst_hbm, zeros_hbm, ones_hbm, out_hbm, dst_v, ones_v, acc_sh):
    c = lax.axis_index("c")
    s = lax.axis_index("s")
    rows = pl.ds(s * _RPS, _RPS)
    pltpu.sync_copy(zeros_hbm.at[rows], acc_sh.at[rows])
    pltpu.sync_copy(ones_hbm, ones_v)
    for ph in range(_NPH):
        pltpu.sync_copy(dst_hbm.at[c, s, ph], dst_v)
        if ph == 0:
            plsc.subcore_barrier()

        @pl.loop(0, _NITER)
        def _(j):
            pltpu.sync_copy(ones_v, acc_sh.at[dst_v.at[j]], add=True)

    plsc.subcore_barrier()
    pltpu.sync_copy(acc_sh.at[rows], out_hbm.at[c, rows])


@functools.partial(
    pl.kernel,
    out_type=jax.ShapeDtypeStruct((_NC, _NPAD, _D), jnp.float32),
    mesh=_mesh,
    scratch_types=[
        pltpu.VMEM((_NITER, _K), jnp.int32),
        pltpu.VMEM((_NITER, _K), jnp.int32),
        pltpu.VMEM((_K, _D), jnp.float32),
        pltpu.VMEM((_K, _D), jnp.float32),
        pltpu.VMEM_SHARED((_NPAD, _D), jnp.float32),
        pltpu.SemaphoreType.DMA,
        pltpu.SemaphoreType.DMA,
        pltpu.SemaphoreType.DMA,
        pltpu.SemaphoreType.DMA,
    ],
)
def _msg_kernel(hs_hbm, src_hbm, dst_hbm, zeros_hbm, out_hbm,
                src_v, dst_v, b0, b1, acc_sh, sg0, sg1, ss0, ss1):
    c = lax.axis_index("c")
    s = lax.axis_index("s")
    rows = pl.ds(s * _RPS, _RPS)
    pltpu.sync_copy(zeros_hbm.at[rows], acc_sh.at[rows])

    first = [True]

    def _phase(ph):
        pltpu.sync_copy(src_hbm.at[c, s, ph], src_v)
        pltpu.sync_copy(dst_hbm.at[c, s, ph], dst_v)
        if first[0]:
            plsc.subcore_barrier()
            first[0] = False
        pltpu.async_copy(hs_hbm.at[src_v.at[0]], b0, sg0)
        pltpu.async_copy(hs_hbm.at[src_v.at[1]], b1, sg1)

        @pl.loop(0, _NITER, step=2)
        def _(j):
            pltpu.make_async_copy(hs_hbm.at[src_v.at[j]], b0, sg0).wait()
            pltpu.async_copy(b0, acc_sh.at[dst_v.at[j]], ss0, add=True)
            pltpu.make_async_copy(hs_hbm.at[src_v.at[j + 1]], b1, sg1).wait()
            pltpu.async_copy(b1, acc_sh.at[dst_v.at[j + 1]], ss1, add=True)

            @pl.when(j + 2 < _NITER)
            def _():
                pltpu.make_async_copy(b0, acc_sh.at[dst_v.at[j]], ss0).wait()
                pltpu.async_copy(hs_hbm.at[src_v.at[j + 2]], b0, sg0)
                pltpu.make_async_copy(b1, acc_sh.at[dst_v.at[j + 1]], ss1).wait()
                pltpu.async_copy(hs_hbm.at[src_v.at[j + 3]], b1, sg1)

        pltpu.make_async_copy(b0, acc_sh.at[dst_v.at[0]], ss0).wait()
        pltpu.make_async_copy(b1, acc_sh.at[dst_v.at[1]], ss1).wait()

    for ph in range(_NPH):
        _phase(ph)
    plsc.subcore_barrier()
    pltpu.sync_copy(acc_sh.at[rows], out_hbm.at[c, rows])


def _dinv_from_degp(degp):
    deg = degp[0, :, 0:1] + degp[1, :, 0:1]
    return lax.rsqrt(jnp.maximum(deg, 1.0))


def _tc_matmul_body(x_ref, w_ref, o_ref):
    o_ref[...] = jnp.dot(x_ref[...], w_ref[...],
                         preferred_element_type=jnp.float32)


_tc_matmul = pl.pallas_call(
    _tc_matmul_body,
    out_shape=jax.ShapeDtypeStruct((_NPAD, _D), jnp.float32),
)


def _tc_scale_body(hm_ref, degp_ref, o_ref):
    o_ref[...] = hm_ref[...] * _dinv_from_degp(degp_ref[...])


_tc_scale = pl.pallas_call(
    _tc_scale_body,
    out_shape=jax.ShapeDtypeStruct((_NPAD, _D), jnp.float32),
)


def _bn_relu(p_ref, degp_ref, b_ref, g_ref, be_ref):
    dinv = _dinv_from_degp(degp_ref[...])
    y = (p_ref[0] + p_ref[1]) * dinv + b_ref[...]
    mask = lax.broadcasted_iota(jnp.int32, (_NPAD, 1), 0) < _N
    ym = jnp.where(mask, y, 0.0)
    mu = jnp.sum(ym, axis=0, keepdims=True) * (1.0 / _N)
    d2 = jnp.where(mask, y - mu, 0.0)
    var = jnp.sum(d2 * d2, axis=0, keepdims=True) * (1.0 / _N)
    h = (y - mu) * lax.rsqrt(var + 1e-5) * g_ref[...] + be_ref[...]
    h = jnp.maximum(h, 0.0)
    return jnp.where(mask, h, 0.0), dinv


def _tc_layer_body(p_ref, degp_ref, b_ref, g_ref, be_ref, w_ref, o_ref):
    h, dinv = _bn_relu(p_ref, degp_ref, b_ref, g_ref, be_ref)
    o_ref[...] = jnp.dot(h * dinv, w_ref[...],
                         preferred_element_type=jnp.float32)


_tc_layer = pl.pallas_call(
    _tc_layer_body,
    out_shape=jax.ShapeDtypeStruct((_NPAD, _D), jnp.float32),
)


def _tc_head_body(p_ref, degp_ref, b_ref, g_ref, be_ref, batch_ref,
                  fw1_ref, fb1_ref, fw2_ref, fb2_ref, o_ref):
    h, _ = _bn_relu(p_ref, degp_ref, b_ref, g_ref, be_ref)
    gi = lax.broadcasted_iota(jnp.int32, (_G, 1), 0)
    oh = (batch_ref[...] == gi).astype(jnp.float32)
    pooled_sum = jax.lax.dot_general(
        oh, h, (((1,), (0,)), ((), ())),
        preferred_element_type=jnp.float32)
    counts = jnp.sum(oh, axis=1, keepdims=True)
    pooled = pooled_sum / jnp.maximum(counts, 1.0)
    z = jnp.maximum(
        jnp.dot(pooled, fw1_ref[...], preferred_element_type=jnp.float32)
        + fb1_ref[...], 0.0)
    o_ref[...] = jnp.dot(z, fw2_ref[...],
                         preferred_element_type=jnp.float32) + fb2_ref[...]


_tc_head = pl.pallas_call(
    _tc_head_body,
    out_shape=jax.ShapeDtypeStruct((_G, _C), jnp.float32),
)


def kernel(x, edge_index, batch, W1, b1, g1, be1, W2, b2, g2, be2,
           W3, b3, g3, be3, fW1, fb1, fW2, fb2):
    f32 = jnp.float32
    loop = jnp.arange(_N, dtype=edge_index.dtype)
    pad = _EPAD - _EP
    spare = _N + jnp.arange(pad, dtype=edge_index.dtype) % (_NPAD - _N)
    src = jnp.concatenate([edge_index[0], loop, spare])
    dst = jnp.concatenate([edge_index[1], loop, spare])
    src_p = src.reshape(_NC, _NS, _NPH, _NITER, _K)
    dst_p = dst.reshape(_NC, _NS, _NPH, _NITER, _K)

    zeros_d = jnp.zeros((_NPAD, _D), f32)
    ones_d = jnp.ones((_K, _D), f32)
    xp = jnp.pad(x, ((0, _NPAD - _N), (0, 0)))
    batch_p = jnp.pad(batch, (0, _NPAD - _N),
                      constant_values=_G).reshape(1, _NPAD)
    row = lambda v: v.reshape(1, -1)

    degp = _deg_kernel(dst_p, zeros_d, ones_d)
    hm1 = _tc_matmul(xp, W1)
    hs1 = _tc_scale(hm1, degp)
    p1 = _msg_kernel(hs1, src_p, dst_p, zeros_d)
    hs2 = _tc_layer(p1, degp, row(b1), row(g1), row(be1), W2)
    p2 = _msg_kernel(hs2, src_p, dst_p, zeros_d)
    hs3 = _tc_layer(p2, degp, row(b2), row(g2), row(be2), W3)
    p3 = _msg_kernel(hs3, src_p, dst_p, zeros_d)
    return _tc_head(p3, degp, row(b3), row(g3), row(be3), batch_p,
                    fW1, row(fb1), fW2, row(fb2))


# speedup vs baseline: 1.0634x; 1.0634x over previous
import functools

import jax
import jax.numpy as jnp
from jax import lax
from jax.experimental import pallas as pl
from jax.experimental.pallas import tpu as pltpu
from jax.experimental.pallas import tpu_sc as plsc

_N = 10000
_E = 320000
_D = 128
_G = 64
_C = 16

_NC = 2
_NS = 16
_K = 128

_NPAD = 10112
_RPS = _NPAD // _NS

_EP = _E + _N
_NPH = 2
_NITER = 42
_EPAD = _NC * _NS * _NPH * _NITER * _K

_mesh = plsc.VectorSubcoreMesh(core_axis_name="c", subcore_axis_name="s")


@functools.partial(
    pl.kernel,
    out_type=jax.ShapeDtypeStruct((_NC, _NPAD, _D), jnp.float32),
    mesh=_mesh,
    scratch_types=[
        pltpu.VMEM((_NITER, _K), jnp.int32),
        pltpu.VMEM((_K, _D), jnp.float32),
        pltpu.VMEM_SHARED((_NPAD, _D), jnp.float32),
    ],
)
def _deg_kernel(dst_hbm, zeros_hbm, ones_hbm, out_hbm, dst_v, ones_v, acc_sh):
    c = lax.axis_index("c")
    s = lax.axis_index("s")
    rows = pl.ds(s * _RPS, _RPS)
    pltpu.sync_copy(zeros_hbm.at[rows], acc_sh.at[rows])
    pltpu.sync_copy(ones_hbm, ones_v)
    for ph in range(_NPH):
        pltpu.sync_copy(dst_hbm.at[c, s, ph], dst_v)
        if ph == 0:
            plsc.subcore_barrier()

        @pl.loop(0, _NITER)
        def _(j):
            pltpu.sync_copy(ones_v, acc_sh.at[dst_v.at[j]], add=True)

    plsc.subcore_barrier()
    pltpu.sync_copy(acc_sh.at[rows], out_hbm.at[c, rows])


@functools.partial(
    pl.kernel,
    out_type=jax.ShapeDtypeStruct((_NC, _NPAD, _D), jnp.float32),
    mesh=_mesh,
    scratch_types=[
        pltpu.VMEM((_NITER, _K), jnp.int32),
        pltpu.VMEM((_NITER, _K), jnp.int32),
        pltpu.VMEM((_K, _D), jnp.float32),
        pltpu.VMEM((_K, _D), jnp.float32),
        pltpu.VMEM_SHARED((_NPAD, _D), jnp.float32),
        pltpu.SemaphoreType.DMA,
        pltpu.SemaphoreType.DMA,
        pltpu.SemaphoreType.DMA,
        pltpu.SemaphoreType.DMA,
    ],
)
def _msg_kernel(hs_hbm, src_hbm, dst_hbm, zeros_hbm, out_hbm,
                src_v, dst_v, b0, b1, acc_sh, sg0, sg1, ss0, ss1):
    c = lax.axis_index("c")
    s = lax.axis_index("s")
    rows = pl.ds(s * _RPS, _RPS)
    pltpu.sync_copy(zeros_hbm.at[rows], acc_sh.at[rows])

    first = [True]

    def _phase(ph):
        pltpu.sync_copy(src_hbm.at[c, s, ph], src_v)
        pltpu.sync_copy(dst_hbm.at[c, s, ph], dst_v)
        if first[0]:
            plsc.subcore_barrier()
            first[0] = False
        @pl.loop(0, _NITER, step=2)
        def _(j):
            @pl.when(j > 0)
            def _():
                pltpu.make_async_copy(b0, acc_sh.at[dst_v.at[j]], ss0).wait()
            pltpu.sync_copy(hs_hbm.at[src_v.at[j]], b0)
            pltpu.async_copy(b0, acc_sh.at[dst_v.at[j]], ss0, add=True)

            @pl.when(j > 0)
            def _():
                pltpu.make_async_copy(b1, acc_sh.at[dst_v.at[j]], ss1).wait()
            pltpu.sync_copy(hs_hbm.at[src_v.at[j + 1]], b1)
            pltpu.async_copy(b1, acc_sh.at[dst_v.at[j + 1]], ss1, add=True)

        pltpu.make_async_copy(b0, acc_sh.at[dst_v.at[0]], ss0).wait()
        pltpu.make_async_copy(b1, acc_sh.at[dst_v.at[1]], ss1).wait()

    for ph in range(_NPH):
        _phase(ph)
    plsc.subcore_barrier()
    pltpu.sync_copy(acc_sh.at[rows], out_hbm.at[c, rows])


def _dinv_from_degp(degt):
    deg = degt[:, 0:1] + degt[:, 1:2]
    return lax.rsqrt(jnp.maximum(deg, 1.0))


def _tc_matmul_body(x_ref, w_ref, o_ref):
    o_ref[...] = jnp.dot(x_ref[...], w_ref[...],
                         preferred_element_type=jnp.float32)


_tc_matmul = pl.pallas_call(
    _tc_matmul_body,
    out_shape=jax.ShapeDtypeStruct((_NPAD, _D), jnp.float32),
)


def _tc_scale_body(hm_ref, degp_ref, o_ref):
    o_ref[...] = hm_ref[...] * _dinv_from_degp(degp_ref[...])


_tc_scale = pl.pallas_call(
    _tc_scale_body,
    out_shape=jax.ShapeDtypeStruct((_NPAD, _D), jnp.float32),
)


def _bn_relu(p_ref, degp_ref, b_ref, g_ref, be_ref):
    dinv = _dinv_from_degp(degp_ref[...])
    y = (p_ref[0] + p_ref[1]) * dinv + b_ref[...]
    mask = lax.broadcasted_iota(jnp.int32, (_NPAD, 1), 0) < _N
    ym = jnp.where(mask, y, 0.0)
    mu = jnp.sum(ym, axis=0, keepdims=True) * (1.0 / _N)
    d2 = jnp.where(mask, y - mu, 0.0)
    var = jnp.sum(d2 * d2, axis=0, keepdims=True) * (1.0 / _N)
    h = (y - mu) * lax.rsqrt(var + 1e-5) * g_ref[...] + be_ref[...]
    h = jnp.maximum(h, 0.0)
    return jnp.where(mask, h, 0.0), dinv


def _tc_layer_body(p_ref, degp_ref, b_ref, g_ref, be_ref, w_ref, o_ref):
    h, dinv = _bn_relu(p_ref, degp_ref, b_ref, g_ref, be_ref)
    o_ref[...] = jnp.dot(h * dinv, w_ref[...],
                         preferred_element_type=jnp.float32)


_tc_layer = pl.pallas_call(
    _tc_layer_body,
    out_shape=jax.ShapeDtypeStruct((_NPAD, _D), jnp.float32),
)


def _tc_head_body(p_ref, degp_ref, b_ref, g_ref, be_ref, batch_ref,
                  fw1_ref, fb1_ref, fw2_ref, fb2_ref, o_ref):
    h, _ = _bn_relu(p_ref, degp_ref, b_ref, g_ref, be_ref)
    gi = lax.broadcasted_iota(jnp.int32, (_G, 1), 0)
    oh = (batch_ref[...] == gi).astype(jnp.float32)
    pooled_sum = jax.lax.dot_general(
        oh, h, (((1,), (0,)), ((), ())),
        preferred_element_type=jnp.float32)
    counts = jnp.sum(oh, axis=1, keepdims=True)
    pooled = pooled_sum / jnp.maximum(counts, 1.0)
    z = jnp.maximum(
        jnp.dot(pooled, fw1_ref[...], preferred_element_type=jnp.float32)
        + fb1_ref[...], 0.0)
    o_ref[...] = jnp.dot(z, fw2_ref[...],
                         preferred_element_type=jnp.float32) + fb2_ref[...]


_tc_head = pl.pallas_call(
    _tc_head_body,
    out_shape=jax.ShapeDtypeStruct((_G, _C), jnp.float32),
)


def kernel(x, edge_index, batch, W1, b1, g1, be1, W2, b2, g2, be2,
           W3, b3, g3, be3, fW1, fb1, fW2, fb2):
    f32 = jnp.float32
    loop = jnp.arange(_N, dtype=edge_index.dtype)
    pad = _EPAD - _EP
    spare = _N + jnp.arange(pad, dtype=edge_index.dtype) % (_NPAD - _N)
    src = jnp.concatenate([edge_index[0], loop, spare])
    dst = jnp.concatenate([edge_index[1], loop, spare])
    src_p = src.reshape(_NC, _NS, _NPH, _NITER, _K)
    dst_p = dst.reshape(_NC, _NS, _NPH, _NITER, _K)

    zeros_d = jnp.zeros((_NPAD, _D), f32)
    ones_d = jnp.ones((_K, _D), f32)
    xp = jnp.pad(x, ((0, _NPAD - _N), (0, 0)))
    batch_p = jnp.pad(batch, (0, _NPAD - _N),
                      constant_values=_G).reshape(1, _NPAD)
    row = lambda v: v.reshape(1, -1)

    degp = _deg_kernel(dst_p, zeros_d, ones_d)
    degt = jnp.stack([degp[0, :, 0], degp[1, :, 0]], axis=1)
    hm1 = _tc_matmul(xp, W1)
    hs1 = _tc_scale(hm1, degt)
    p1 = _msg_kernel(hs1, src_p, dst_p, zeros_d)
    hs2 = _tc_layer(p1, degt, row(b1), row(g1), row(be1), W2)
    p2 = _msg_kernel(hs2, src_p, dst_p, zeros_d)
    hs3 = _tc_layer(p2, degt, row(b2), row(g2), row(be2), W3)
    p3 = _msg_kernel(hs3, src_p, dst_p, zeros_d)
    return _tc_head(p3, degt, row(b3), row(g3), row(be3), batch_p,
                    fW1, row(fb1), fW2, row(fb2))


# speedup vs baseline: 1.0759x; 1.0118x over previous
import functools

import jax
import jax.numpy as jnp
from jax import lax
from jax.experimental import pallas as pl
from jax.experimental.pallas import tpu as pltpu
from jax.experimental.pallas import tpu_sc as plsc

_N = 10000
_E = 320000
_D = 128
_G = 64
_C = 16

_NC = 2
_NS = 16
_K = 128

_NPAD = 10112
_RPS = _NPAD // _NS

_EP = _E + _N
_NPH = 2
_NITER = 42
_EPAD = _NC * _NS * _NPH * _NITER * _K

_mesh = plsc.VectorSubcoreMesh(core_axis_name="c", subcore_axis_name="s")


@functools.partial(
    pl.kernel,
    out_type=jax.ShapeDtypeStruct((_NC, _NPAD, _D), jnp.float32),
    mesh=_mesh,
    scratch_types=[
        pltpu.VMEM((_NITER, _K), jnp.int32),
        pltpu.VMEM((_K, _D), jnp.float32),
        pltpu.VMEM_SHARED((_NPAD, _D), jnp.float32),
    ],
)
def _deg_kernel(dst_hbm, zeros_hbm, ones_hbm, out_hbm, dst_v, ones_v, acc_sh):
    c = lax.axis_index("c")
    s = lax.axis_index("s")
    rows = pl.ds(s * _RPS, _RPS)
    pltpu.sync_copy(zeros_hbm.at[rows], acc_sh.at[rows])
    pltpu.sync_copy(ones_hbm, ones_v)
    for ph in range(_NPH):
        pltpu.sync_copy(dst_hbm.at[c, s, ph], dst_v)
        if ph == 0:
            plsc.subcore_barrier()

        @pl.loop(0, _NITER)
        def _(j):
            pltpu.sync_copy(ones_v, acc_sh.at[dst_v.at[j]], add=True)

    plsc.subcore_barrier()
    pltpu.sync_copy(acc_sh.at[rows], out_hbm.at[c, rows])


@functools.partial(
    pl.kernel,
    out_type=jax.ShapeDtypeStruct((_NC, _NPAD, _D), jnp.float32),
    mesh=_mesh,
    scratch_types=[
        pltpu.VMEM((_NITER, _K), jnp.int32),
        pltpu.VMEM((_NITER, _K), jnp.int32),
        pltpu.VMEM((_K, _D), jnp.float32),
        pltpu.VMEM((_K, _D), jnp.float32),
        pltpu.VMEM_SHARED((_NPAD, _D), jnp.float32),
        pltpu.SemaphoreType.DMA,
        pltpu.SemaphoreType.DMA,
        pltpu.SemaphoreType.DMA,
        pltpu.SemaphoreType.DMA,
    ],
)
def _msg_kernel(hs_hbm, src_hbm, dst_hbm, zeros_hbm, out_hbm,
                src_v, dst_v, b0, b1, acc_sh, sg0, sg1, ss0, ss1):
    c = lax.axis_index("c")
    s = lax.axis_index("s")
    rows = pl.ds(s * _RPS, _RPS)
    pltpu.sync_copy(zeros_hbm.at[rows], acc_sh.at[rows])

    first = [True]

    def _phase(ph):
        pltpu.sync_copy(src_hbm.at[c, s, ph], src_v)
        pltpu.sync_copy(dst_hbm.at[c, s, ph], dst_v)
        if first[0]:
            plsc.subcore_barrier()
            first[0] = False
        @pl.loop(0, _NITER, step=2)
        def _(j):
            @pl.when(j > 0)
            def _():
                pltpu.make_async_copy(b0, acc_sh.at[dst_v.at[j]], ss0).wait()
            pltpu.sync_copy(hs_hbm.at[src_v.at[j]], b0)
            pltpu.async_copy(b0, acc_sh.at[dst_v.at[j]], ss0, add=True)

            @pl.when(j > 0)
            def _():
                pltpu.make_async_copy(b1, acc_sh.at[dst_v.at[j]], ss1).wait()
            pltpu.sync_copy(hs_hbm.at[src_v.at[j + 1]], b1)
            pltpu.async_copy(b1, acc_sh.at[dst_v.at[j + 1]], ss1, add=True)

        pltpu.make_async_copy(b0, acc_sh.at[dst_v.at[0]], ss0).wait()
        pltpu.make_async_copy(b1, acc_sh.at[dst_v.at[1]], ss1).wait()

    for ph in range(_NPH):
        _phase(ph)
    plsc.subcore_barrier()
    pltpu.sync_copy(acc_sh.at[rows], out_hbm.at[c, rows])


def _dinv_from_degp(degt):
    deg = degt[:, 0:1] + degt[:, 1:2]
    return lax.rsqrt(jnp.maximum(deg, 1.0))


def _tc_matmul_body(x_ref, w_ref, o_ref):
    o_ref[...] = jnp.dot(x_ref[...], w_ref[...],
                         preferred_element_type=jnp.float32)


_tc_matmul = pl.pallas_call(
    _tc_matmul_body,
    out_shape=jax.ShapeDtypeStruct((_NPAD, _D), jnp.float32),
)


def _tc_scale_body(hm_ref, degp_ref, o_ref, degt_ref):
    degt = jnp.concatenate(
        [degp_ref[0, :, 0:1], degp_ref[1, :, 0:1]], axis=1)
    degt_ref[...] = degt
    o_ref[...] = hm_ref[...] * _dinv_from_degp(degt)


_tc_scale = pl.pallas_call(
    _tc_scale_body,
    out_shape=(jax.ShapeDtypeStruct((_NPAD, _D), jnp.float32),
               jax.ShapeDtypeStruct((_NPAD, 2), jnp.float32)),
)


def _bn_relu(p_ref, degp_ref, b_ref, g_ref, be_ref):
    dinv = _dinv_from_degp(degp_ref[...])
    y = (p_ref[0] + p_ref[1]) * dinv + b_ref[...]
    mask = lax.broadcasted_iota(jnp.int32, (_NPAD, 1), 0) < _N
    ym = jnp.where(mask, y, 0.0)
    mu = jnp.sum(ym, axis=0, keepdims=True) * (1.0 / _N)
    d2 = jnp.where(mask, y - mu, 0.0)
    var = jnp.sum(d2 * d2, axis=0, keepdims=True) * (1.0 / _N)
    h = (y - mu) * lax.rsqrt(var + 1e-5) * g_ref[...] + be_ref[...]
    h = jnp.maximum(h, 0.0)
    return jnp.where(mask, h, 0.0), dinv


def _tc_layer_body(p_ref, degp_ref, b_ref, g_ref, be_ref, w_ref, o_ref):
    h, dinv = _bn_relu(p_ref, degp_ref, b_ref, g_ref, be_ref)
    o_ref[...] = jnp.dot(h * dinv, w_ref[...],
                         preferred_element_type=jnp.float32)


_tc_layer = pl.pallas_call(
    _tc_layer_body,
    out_shape=jax.ShapeDtypeStruct((_NPAD, _D), jnp.float32),
)


def _tc_head_body(p_ref, degp_ref, b_ref, g_ref, be_ref, batch_ref,
                  fw1_ref, fb1_ref, fw2_ref, fb2_ref, o_ref):
    h, _ = _bn_relu(p_ref, degp_ref, b_ref, g_ref, be_ref)
    gi = lax.broadcasted_iota(jnp.int32, (_G, 1), 0)
    oh = (batch_ref[...] == gi).astype(jnp.float32)
    pooled_sum = jax.lax.dot_general(
        oh, h, (((1,), (0,)), ((), ())),
        preferred_element_type=jnp.float32)
    counts = jnp.sum(oh, axis=1, keepdims=True)
    pooled = pooled_sum / jnp.maximum(counts, 1.0)
    z = jnp.maximum(
        jnp.dot(pooled, fw1_ref[...], preferred_element_type=jnp.float32)
        + fb1_ref[...], 0.0)
    o_ref[...] = jnp.dot(z, fw2_ref[...],
                         preferred_element_type=jnp.float32) + fb2_ref[...]


_tc_head = pl.pallas_call(
    _tc_head_body,
    out_shape=jax.ShapeDtypeStruct((_G, _C), jnp.float32),
)


def kernel(x, edge_index, batch, W1, b1, g1, be1, W2, b2, g2, be2,
           W3, b3, g3, be3, fW1, fb1, fW2, fb2):
    f32 = jnp.float32
    loop = jnp.arange(_N, dtype=edge_index.dtype)
    pad = _EPAD - _EP
    spare = _N + jnp.arange(pad, dtype=edge_index.dtype) % (_NPAD - _N)
    src = jnp.concatenate([edge_index[0], loop, spare])
    dst = jnp.concatenate([edge_index[1], loop, spare])
    src_p = src.reshape(_NC, _NS, _NPH, _NITER, _K)
    dst_p = dst.reshape(_NC, _NS, _NPH, _NITER, _K)

    zeros_d = jnp.zeros((_NPAD, _D), f32)
    ones_d = jnp.ones((_K, _D), f32)
    xp = jnp.pad(x, ((0, _NPAD - _N), (0, 0)))
    batch_p = jnp.pad(batch, (0, _NPAD - _N),
                      constant_values=_G).reshape(1, _NPAD)
    row = lambda v: v.reshape(1, -1)

    degp = _deg_kernel(dst_p, zeros_d, ones_d)
    hm1 = _tc_matmul(xp, W1)
    hs1, degt = _tc_scale(hm1, degp)
    p1 = _msg_kernel(hs1, src_p, dst_p, zeros_d)
    hs2 = _tc_layer(p1, degt, row(b1), row(g1), row(be1), W2)
    p2 = _msg_kernel(hs2, src_p, dst_p, zeros_d)
    hs3 = _tc_layer(p2, degt, row(b2), row(g2), row(be2), W3)
    p3 = _msg_kernel(hs3, src_p, dst_p, zeros_d)
    return _tc_head(p3, degt, row(b3), row(g3), row(be3), batch_p,
                    fW1, row(fb1), fW2, row(fb2))


# speedup vs baseline: 1.1956x; 1.1112x over previous
import dataclasses
import functools

import jax
import jax.numpy as jnp
from jax import lax
from jax.experimental import pallas as pl
from jax.experimental.pallas import tpu as pltpu
from jax.experimental.pallas import tpu_sc as plsc

_N = 10000
_E = 320000
_D = 128
_G = 64
_C = 16

_NC = 2
_NS = 16
_K = 128

_NPAD = 10112
_RPS = _NPAD // _NS

_EP = _E + _N
_NPH = 2
_NITER = 42
_EPAD = _NC * _NS * _NPH * _NITER * _K

_mesh = plsc.VectorSubcoreMesh(core_axis_name="c", subcore_axis_name="s")


@functools.partial(
    pl.kernel,
    out_type=jax.ShapeDtypeStruct((_NC, _NS, _NPAD), jnp.float32),
    mesh=_mesh,
    scratch_types=[
        pltpu.VMEM((_NPH * _NITER, _K), jnp.int32),
        pltpu.VMEM((_NPAD,), jnp.float32),
    ],
    compiler_params=dataclasses.replace(
        pltpu.CompilerParams(), needs_layout_passes=False),
)
def _deg_kernel(dst_hbm, out_hbm, dst_v, hist_v):
    c = lax.axis_index("c")
    s = lax.axis_index("s")
    for ph in range(_NPH):
        pltpu.sync_copy(dst_hbm.at[c, s, ph],
                        dst_v.at[pl.ds(ph * _NITER, _NITER)])

    @pl.loop(0, _NPAD // 16)
    def _(i):
        hist_v[pl.ds(i * 16, 16)] = jnp.zeros((16,), jnp.float32)

    @pl.loop(0, _NPH * _NITER)
    def _(j):
        for i in range(_K // 16):
            vec = dst_v[j, pl.ds(i * 16, 16)]
            cnt, lastm = plsc.scan_count(vec)
            old = plsc.load_gather(hist_v, [vec])
            plsc.store_scatter(hist_v, [vec],
                               old + cnt.astype(jnp.float32), mask=lastm)

    pltpu.sync_copy(hist_v, out_hbm.at[c, s])


@functools.partial(
    pl.kernel,
    out_type=jax.ShapeDtypeStruct((_NC, _NPAD, _D), jnp.float32),
    mesh=_mesh,
    scratch_types=[
        pltpu.VMEM((_NITER, _K), jnp.int32),
        pltpu.VMEM((_NITER, _K), jnp.int32),
        pltpu.VMEM((_K, _D), jnp.float32),
        pltpu.VMEM((_K, _D), jnp.float32),
        pltpu.VMEM_SHARED((_NPAD, _D), jnp.float32),
        pltpu.SemaphoreType.DMA,
        pltpu.SemaphoreType.DMA,
        pltpu.SemaphoreType.DMA,
        pltpu.SemaphoreType.DMA,
    ],
)
def _msg_kernel(hs_hbm, src_hbm, dst_hbm, zeros_hbm, out_hbm,
                src_v, dst_v, b0, b1, acc_sh, sg0, sg1, ss0, ss1):
    c = lax.axis_index("c")
    s = lax.axis_index("s")
    rows = pl.ds(s * _RPS, _RPS)
    pltpu.sync_copy(zeros_hbm.at[rows], acc_sh.at[rows])

    first = [True]

    def _phase(ph):
        pltpu.sync_copy(src_hbm.at[c, s, ph], src_v)
        pltpu.sync_copy(dst_hbm.at[c, s, ph], dst_v)
        if first[0]:
            plsc.subcore_barrier()
            first[0] = False
        @pl.loop(0, _NITER, step=2)
        def _(j):
            @pl.when(j > 0)
            def _():
                pltpu.make_async_copy(b0, acc_sh.at[dst_v.at[j]], ss0).wait()
            pltpu.sync_copy(hs_hbm.at[src_v.at[j]], b0)
            pltpu.async_copy(b0, acc_sh.at[dst_v.at[j]], ss0, add=True)

            @pl.when(j > 0)
            def _():
                pltpu.make_async_copy(b1, acc_sh.at[dst_v.at[j]], ss1).wait()
            pltpu.sync_copy(hs_hbm.at[src_v.at[j + 1]], b1)
            pltpu.async_copy(b1, acc_sh.at[dst_v.at[j + 1]], ss1, add=True)

        pltpu.make_async_copy(b0, acc_sh.at[dst_v.at[0]], ss0).wait()
        pltpu.make_async_copy(b1, acc_sh.at[dst_v.at[1]], ss1).wait()

    for ph in range(_NPH):
        _phase(ph)
    plsc.subcore_barrier()
    pltpu.sync_copy(acc_sh.at[rows], out_hbm.at[c, rows])


def _dinv_from_degp(degt):
    deg = degt[:, 0:1] + degt[:, 1:2]
    return lax.rsqrt(jnp.maximum(deg, 1.0))


def _tc_matmul_body(x_ref, w_ref, o_ref):
    o_ref[...] = jnp.dot(x_ref[...], w_ref[...],
                         preferred_element_type=jnp.float32)


_tc_matmul = pl.pallas_call(
    _tc_matmul_body,
    out_shape=jax.ShapeDtypeStruct((_NPAD, _D), jnp.float32),
)


def _tc_scale_body(hm_ref, degh_ref, o_ref, degt_ref):
    deg = jnp.sum(degh_ref[...], axis=1, keepdims=True)
    degt = jnp.concatenate([deg, jnp.zeros_like(deg)], axis=1)
    degt_ref[...] = degt
    o_ref[...] = hm_ref[...] * _dinv_from_degp(degt)


_tc_scale = pl.pallas_call(
    _tc_scale_body,
    out_shape=(jax.ShapeDtypeStruct((_NPAD, _D), jnp.float32),
               jax.ShapeDtypeStruct((_NPAD, 2), jnp.float32)),
)


def _bn_relu(p_ref, degp_ref, b_ref, g_ref, be_ref):
    dinv = _dinv_from_degp(degp_ref[...])
    y = (p_ref[0] + p_ref[1]) * dinv + b_ref[...]
    mask = lax.broadcasted_iota(jnp.int32, (_NPAD, 1), 0) < _N
    ym = jnp.where(mask, y, 0.0)
    mu = jnp.sum(ym, axis=0, keepdims=True) * (1.0 / _N)
    d2 = jnp.where(mask, y - mu, 0.0)
    var = jnp.sum(d2 * d2, axis=0, keepdims=True) * (1.0 / _N)
    h = (y - mu) * lax.rsqrt(var + 1e-5) * g_ref[...] + be_ref[...]
    h = jnp.maximum(h, 0.0)
    return jnp.where(mask, h, 0.0), dinv


def _tc_layer_body(p_ref, degp_ref, b_ref, g_ref, be_ref, w_ref, o_ref):
    h, dinv = _bn_relu(p_ref, degp_ref, b_ref, g_ref, be_ref)
    o_ref[...] = jnp.dot(h * dinv, w_ref[...],
                         preferred_element_type=jnp.float32)


_tc_layer = pl.pallas_call(
    _tc_layer_body,
    out_shape=jax.ShapeDtypeStruct((_NPAD, _D), jnp.float32),
)


def _tc_head_body(p_ref, degp_ref, b_ref, g_ref, be_ref, batch_ref,
                  fw1_ref, fb1_ref, fw2_ref, fb2_ref, o_ref):
    h, _ = _bn_relu(p_ref, degp_ref, b_ref, g_ref, be_ref)
    gi = lax.broadcasted_iota(jnp.int32, (_G, 1), 0)
    oh = (batch_ref[...] == gi).astype(jnp.float32)
    pooled_sum = jax.lax.dot_general(
        oh, h, (((1,), (0,)), ((), ())),
        preferred_element_type=jnp.float32)
    counts = jnp.sum(oh, axis=1, keepdims=True)
    pooled = pooled_sum / jnp.maximum(counts, 1.0)
    z = jnp.maximum(
        jnp.dot(pooled, fw1_ref[...], preferred_element_type=jnp.float32)
        + fb1_ref[...], 0.0)
    o_ref[...] = jnp.dot(z, fw2_ref[...],
                         preferred_element_type=jnp.float32) + fb2_ref[...]


_tc_head = pl.pallas_call(
    _tc_head_body,
    out_shape=jax.ShapeDtypeStruct((_G, _C), jnp.float32),
)


def kernel(x, edge_index, batch, W1, b1, g1, be1, W2, b2, g2, be2,
           W3, b3, g3, be3, fW1, fb1, fW2, fb2):
    f32 = jnp.float32
    loop = jnp.arange(_N, dtype=edge_index.dtype)
    pad = _EPAD - _EP
    spare = _N + jnp.arange(pad, dtype=edge_index.dtype) % (_NPAD - _N)
    src = jnp.concatenate([edge_index[0], loop, spare])
    dst = jnp.concatenate([edge_index[1], loop, spare])
    src_p = src.reshape(_NC, _NS, _NPH, _NITER, _K)
    dst_p = dst.reshape(_NC, _NS, _NPH, _NITER, _K)

    zeros_d = jnp.zeros((_NPAD, _D), f32)
    xp = jnp.pad(x, ((0, _NPAD - _N), (0, 0)))
    batch_p = jnp.pad(batch, (0, _NPAD - _N),
                      constant_values=_G).reshape(1, _NPAD)
    row = lambda v: v.reshape(1, -1)

    degh = _deg_kernel(dst_p)
    degh_t = degh.reshape(_NC * _NS, _NPAD).T
    hm1 = _tc_matmul(xp, W1)
    hs1, degt = _tc_scale(hm1, degh_t)
    p1 = _msg_kernel(hs1, src_p, dst_p, zeros_d)
    hs2 = _tc_layer(p1, degt, row(b1), row(g1), row(be1), W2)
    p2 = _msg_kernel(hs2, src_p, dst_p, zeros_d)
    hs3 = _tc_layer(p2, degt, row(b2), row(g2), row(be2), W3)
    p3 = _msg_kernel(hs3, src_p, dst_p, zeros_d)
    return _tc_head(p3, degt, row(b3), row(g3), row(be3), batch_p,
                    fW1, row(fb1), fW2, row(fb2))


# speedup vs baseline: 1.2250x; 1.0246x over previous
import dataclasses
import functools

import jax
import jax.numpy as jnp
from jax import lax
from jax.experimental import pallas as pl
from jax.experimental.pallas import tpu as pltpu
from jax.experimental.pallas import tpu_sc as plsc

_N = 10000
_E = 320000
_D = 128
_G = 64
_C = 16

_NC = 2
_NS = 16
_K = 128

_NPAD = 10112
_RPS = _NPAD // _NS

_EP = _E + _N
_NWIN = 82
_PHASES = (48, 34)
_EPAD = _NC * _NS * _NWIN * _K

_mesh = plsc.VectorSubcoreMesh(core_axis_name="c", subcore_axis_name="s")


@functools.partial(
    pl.kernel,
    out_type=jax.ShapeDtypeStruct((_NC, _NS, _NPAD), jnp.float32),
    mesh=_mesh,
    scratch_types=[
        pltpu.VMEM((_NWIN, _K), jnp.int32),
        pltpu.VMEM((_NPAD,), jnp.float32),
    ],
    compiler_params=dataclasses.replace(
        pltpu.CompilerParams(), needs_layout_passes=False),
)
def _deg_kernel(dst_hbm, out_hbm, dst_v, hist_v):
    c = lax.axis_index("c")
    s = lax.axis_index("s")
    pltpu.sync_copy(dst_hbm.at[c, s], dst_v)

    @pl.loop(0, _NPAD // 16)
    def _(i):
        hist_v[pl.ds(i * 16, 16)] = jnp.zeros((16,), jnp.float32)

    @pl.loop(0, _NWIN)
    def _(j):
        for i in range(_K // 16):
            vec = dst_v[j, pl.ds(i * 16, 16)]
            cnt, lastm = plsc.scan_count(vec)
            old = plsc.load_gather(hist_v, [vec])
            plsc.store_scatter(hist_v, [vec],
                               old + cnt.astype(jnp.float32), mask=lastm)

    pltpu.sync_copy(hist_v, out_hbm.at[c, s])


@functools.partial(
    pl.kernel,
    out_type=jax.ShapeDtypeStruct((_NC, _NPAD, _D), jnp.float32),
    mesh=_mesh,
    scratch_types=[
        pltpu.VMEM((_PHASES[0], _K), jnp.int32),
        pltpu.VMEM((_PHASES[0], _K), jnp.int32),
        pltpu.VMEM((_K, _D), jnp.float32),
        pltpu.VMEM((_K, _D), jnp.float32),
        pltpu.VMEM_SHARED((_NPAD, _D), jnp.float32),
        pltpu.SemaphoreType.DMA,
        pltpu.SemaphoreType.DMA,
        pltpu.SemaphoreType.DMA,
        pltpu.SemaphoreType.DMA,
    ],
)
def _msg_kernel(hs_hbm, src_hbm, dst_hbm, zeros_hbm, out_hbm,
                src_v, dst_v, b0, b1, acc_sh, sg0, sg1, ss0, ss1):
    c = lax.axis_index("c")
    s = lax.axis_index("s")
    rows = pl.ds(s * _RPS, _RPS)
    zc = pltpu.async_copy(zeros_hbm.at[rows], acc_sh.at[rows], ss0)

    off = 0
    for ph, n in enumerate(_PHASES):
        pltpu.sync_copy(src_hbm.at[c, s, pl.ds(off, n)],
                        src_v.at[pl.ds(0, n)])
        pltpu.sync_copy(dst_hbm.at[c, s, pl.ds(off, n)],
                        dst_v.at[pl.ds(0, n)])
        off += n
        if ph == 0:
            zc.wait()
            plsc.subcore_barrier()

        @pl.loop(0, n, step=2)
        def _(j):
            @pl.when(j > 0)
            def _():
                pltpu.make_async_copy(b0, acc_sh.at[dst_v.at[j]], ss0).wait()
            pltpu.sync_copy(hs_hbm.at[src_v.at[j]], b0)
            pltpu.async_copy(b0, acc_sh.at[dst_v.at[j]], ss0, add=True)

            @pl.when(j > 0)
            def _():
                pltpu.make_async_copy(b1, acc_sh.at[dst_v.at[j]], ss1).wait()
            pltpu.sync_copy(hs_hbm.at[src_v.at[j + 1]], b1)
            pltpu.async_copy(b1, acc_sh.at[dst_v.at[j + 1]], ss1, add=True)

        pltpu.make_async_copy(b0, acc_sh.at[dst_v.at[0]], ss0).wait()
        pltpu.make_async_copy(b1, acc_sh.at[dst_v.at[1]], ss1).wait()

    plsc.subcore_barrier()
    pltpu.sync_copy(acc_sh.at[rows], out_hbm.at[c, rows])


def _dinv_from_degp(degt):
    deg = degt[:, 0:1] + degt[:, 1:2]
    return lax.rsqrt(jnp.maximum(deg, 1.0))


def _tc_matmul_body(x_ref, w_ref, o_ref):
    o_ref[...] = jnp.dot(x_ref[...], w_ref[...],
                         preferred_element_type=jnp.float32)


_tc_matmul = pl.pallas_call(
    _tc_matmul_body,
    out_shape=jax.ShapeDtypeStruct((_NPAD, _D), jnp.float32),
)


def _tc_scale_body(hm_ref, degh_ref, o_ref, degt_ref):
    deg = jnp.sum(degh_ref[...], axis=1, keepdims=True)
    degt = jnp.concatenate([deg, jnp.zeros_like(deg)], axis=1)
    degt_ref[...] = degt
    o_ref[...] = hm_ref[...] * _dinv_from_degp(degt)


_tc_scale = pl.pallas_call(
    _tc_scale_body,
    out_shape=(jax.ShapeDtypeStruct((_NPAD, _D), jnp.float32),
               jax.ShapeDtypeStruct((_NPAD, 2), jnp.float32)),
)


def _bn_relu(p_ref, degp_ref, b_ref, g_ref, be_ref):
    dinv = _dinv_from_degp(degp_ref[...])
    y = (p_ref[0] + p_ref[1]) * dinv + b_ref[...]
    mask = lax.broadcasted_iota(jnp.int32, (_NPAD, 1), 0) < _N
    ym = jnp.where(mask, y, 0.0)
    mu = jnp.sum(ym, axis=0, keepdims=True) * (1.0 / _N)
    d2 = jnp.where(mask, y - mu, 0.0)
    var = jnp.sum(d2 * d2, axis=0, keepdims=True) * (1.0 / _N)
    h = (y - mu) * lax.rsqrt(var + 1e-5) * g_ref[...] + be_ref[...]
    h = jnp.maximum(h, 0.0)
    return jnp.where(mask, h, 0.0), dinv


def _tc_layer_body(p_ref, degp_ref, b_ref, g_ref, be_ref, w_ref, o_ref):
    h, dinv = _bn_relu(p_ref, degp_ref, b_ref, g_ref, be_ref)
    o_ref[...] = jnp.dot(h * dinv, w_ref[...],
                         preferred_element_type=jnp.float32)


_tc_layer = pl.pallas_call(
    _tc_layer_body,
    out_shape=jax.ShapeDtypeStruct((_NPAD, _D), jnp.float32),
)


def _tc_head_body(p_ref, degp_ref, b_ref, g_ref, be_ref, batch_ref,
                  fw1_ref, fb1_ref, fw2_ref, fb2_ref, o_ref):
    h, _ = _bn_relu(p_ref, degp_ref, b_ref, g_ref, be_ref)
    gi = lax.broadcasted_iota(jnp.int32, (_G, 1), 0)
    oh = (batch_ref[...] == gi).astype(jnp.float32)
    pooled_sum = jax.lax.dot_general(
        oh, h, (((1,), (0,)), ((), ())),
        preferred_element_type=jnp.float32)
    counts = jnp.sum(oh, axis=1, keepdims=True)
    pooled = pooled_sum / jnp.maximum(counts, 1.0)
    z = jnp.maximum(
        jnp.dot(pooled, fw1_ref[...], preferred_element_type=jnp.float32)
        + fb1_ref[...], 0.0)
    o_ref[...] = jnp.dot(z, fw2_ref[...],
                         preferred_element_type=jnp.float32) + fb2_ref[...]


_tc_head = pl.pallas_call(
    _tc_head_body,
    out_shape=jax.ShapeDtypeStruct((_G, _C), jnp.float32),
)


def kernel(x, edge_index, batch, W1, b1, g1, be1, W2, b2, g2, be2,
           W3, b3, g3, be3, fW1, fb1, fW2, fb2):
    f32 = jnp.float32
    loop = jnp.arange(_N, dtype=edge_index.dtype)
    pad = _EPAD - _EP
    spare = _N + jnp.arange(pad, dtype=edge_index.dtype) % (_NPAD - _N)
    src = jnp.concatenate([edge_index[0], loop, spare])
    dst = jnp.concatenate([edge_index[1], loop, spare])
    src_p = src.reshape(_NC, _NS, _NWIN, _K)
    dst_p = dst.reshape(_NC, _NS, _NWIN, _K)

    zeros_d = jnp.zeros((_NPAD, _D), f32)
    xp = jnp.pad(x, ((0, _NPAD - _N), (0, 0)))
    batch_p = jnp.pad(batch, (0, _NPAD - _N),
                      constant_values=_G).reshape(1, _NPAD)
    row = lambda v: v.reshape(1, -1)

    degh = _deg_kernel(dst_p)
    degh_t = degh.reshape(_NC * _NS, _NPAD).T
    hm1 = _tc_matmul(xp, W1)
    hs1, degt = _tc_scale(hm1, degh_t)
    p1 = _msg_kernel(hs1, src_p, dst_p, zeros_d)
    hs2 = _tc_layer(p1, degt, row(b1), row(g1), row(be1), W2)
    p2 = _msg_kernel(hs2, src_p, dst_p, zeros_d)
    hs3 = _tc_layer(p2, degt, row(b2), row(g2), row(be2), W3)
    p3 = _msg_kernel(hs3, src_p, dst_p, zeros_d)
    return _tc_head(p3, degt, row(b3), row(g3), row(be3), batch_p,
                    fW1, row(fb1), fW2, row(fb2))


# speedup vs baseline: 1.2266x; 1.0013x over previous
import dataclasses
import functools

import jax
import jax.numpy as jnp
from jax import lax
from jax.experimental import pallas as pl
from jax.experimental.pallas import tpu as pltpu
from jax.experimental.pallas import tpu_sc as plsc

_N = 10000
_E = 320000
_D = 128
_G = 64
_C = 16

_NC = 2
_NS = 16
_K = 128

_NPAD = 10112
_RPS = _NPAD // _NS

_EP = _E + _N
_NWIN = 82
_PHASES = (48, 34)
_EPAD = _NC * _NS * _NWIN * _K

_mesh = plsc.VectorSubcoreMesh(core_axis_name="c", subcore_axis_name="s")


@functools.partial(
    pl.kernel,
    out_type=jax.ShapeDtypeStruct((_NC, _NS, _NPAD), jnp.float32),
    mesh=_mesh,
    scratch_types=[
        pltpu.VMEM((_NWIN, _K), jnp.int32),
        pltpu.VMEM((_NPAD,), jnp.float32),
    ],
    compiler_params=dataclasses.replace(
        pltpu.CompilerParams(), needs_layout_passes=False),
)
def _deg_kernel(dst_hbm, out_hbm, dst_v, hist_v):
    c = lax.axis_index("c")
    s = lax.axis_index("s")
    pltpu.sync_copy(dst_hbm.at[c, s], dst_v)

    @pl.loop(0, _NPAD // 16)
    def _(i):
        hist_v[pl.ds(i * 16, 16)] = jnp.zeros((16,), jnp.float32)

    @pl.loop(0, _NWIN)
    def _(j):
        for i in range(_K // 16):
            vec = dst_v[j, pl.ds(i * 16, 16)]
            cnt, lastm = plsc.scan_count(vec)
            old = plsc.load_gather(hist_v, [vec])
            plsc.store_scatter(hist_v, [vec],
                               old + cnt.astype(jnp.float32), mask=lastm)

    pltpu.sync_copy(hist_v, out_hbm.at[c, s])


@functools.partial(
    pl.kernel,
    out_type=jax.ShapeDtypeStruct((_NC, _NPAD, _D), jnp.float32),
    mesh=_mesh,
    scratch_types=[
        pltpu.VMEM((_PHASES[0], _K), jnp.int32),
        pltpu.VMEM((_PHASES[0], _K), jnp.int32),
        pltpu.VMEM((_K, _D), jnp.float32),
        pltpu.VMEM((_K, _D), jnp.float32),
        pltpu.VMEM_SHARED((_NPAD, _D), jnp.float32),
        pltpu.SemaphoreType.DMA,
        pltpu.SemaphoreType.DMA,
        pltpu.SemaphoreType.DMA,
        pltpu.SemaphoreType.DMA,
    ],
)
def _msg_kernel(hs_hbm, src_hbm, dst_hbm, zeros_hbm, out_hbm,
                src_v, dst_v, b0, b1, acc_sh, sg0, sg1, ss0, ss1):
    c = lax.axis_index("c")
    s = lax.axis_index("s")
    rows = pl.ds(s * _RPS, _RPS)
    zc = pltpu.async_copy(zeros_hbm.at[rows], acc_sh.at[rows], ss0)

    off = 0
    for ph, n in enumerate(_PHASES):
        pltpu.sync_copy(src_hbm.at[c, s, pl.ds(off, n)], src_v.at[pl.ds(0, n)])
        pltpu.sync_copy(dst_hbm.at[c, s, pl.ds(off, n)], dst_v.at[pl.ds(0, n)])
        off += n
        if ph == 0:
            zc.wait()
            plsc.subcore_barrier()

        @pl.loop(0, n, step=2)
        def _(j):
            @pl.when(j > 0)
            def _():
                pltpu.make_async_copy(b0, acc_sh.at[dst_v.at[j]], ss0).wait()
            pltpu.sync_copy(hs_hbm.at[src_v.at[j]], b0)
            pltpu.async_copy(b0, acc_sh.at[dst_v.at[j]], ss0, add=True)

            @pl.when(j > 0)
            def _():
                pltpu.make_async_copy(b1, acc_sh.at[dst_v.at[j]], ss1).wait()
            pltpu.sync_copy(hs_hbm.at[src_v.at[j + 1]], b1)
            pltpu.async_copy(b1, acc_sh.at[dst_v.at[j + 1]], ss1, add=True)

        pltpu.make_async_copy(b0, acc_sh.at[dst_v.at[0]], ss0).wait()
        pltpu.make_async_copy(b1, acc_sh.at[dst_v.at[1]], ss1).wait()

    plsc.subcore_barrier()
    pltpu.sync_copy(acc_sh.at[rows], out_hbm.at[c, rows])


def _dinv_from_degp(degt):
    deg = degt[:, 0:1] + degt[:, 1:2]
    return lax.rsqrt(jnp.maximum(deg, 1.0))


def _tc_matmul_body(x_ref, w_ref, o_ref):
    o_ref[...] = jnp.dot(x_ref[...], w_ref[...],
                         preferred_element_type=jnp.float32)


_tc_matmul = pl.pallas_call(
    _tc_matmul_body,
    out_shape=jax.ShapeDtypeStruct((_NPAD, _D), jnp.float32),
)


def _tc_scale_body(x_ref, w_ref, degh_ref, o_ref, degt_ref):
    deg = jnp.sum(degh_ref[...], axis=1, keepdims=True)
    degt = jnp.concatenate([deg, jnp.zeros_like(deg)], axis=1)
    degt_ref[...] = degt
    hm = jnp.dot(x_ref[...], w_ref[...], preferred_element_type=jnp.float32)
    o_ref[...] = hm * _dinv_from_degp(degt)


_tc_scale = pl.pallas_call(
    _tc_scale_body,
    out_shape=(jax.ShapeDtypeStruct((_NPAD, _D), jnp.float32),
               jax.ShapeDtypeStruct((_NPAD, 2), jnp.float32)),
)


def _bn_relu(p_ref, degp_ref, b_ref, g_ref, be_ref):
    dinv = _dinv_from_degp(degp_ref[...])
    y = (p_ref[0] + p_ref[1]) * dinv + b_ref[...]
    mask = lax.broadcasted_iota(jnp.int32, (_NPAD, 1), 0) < _N
    ym = jnp.where(mask, y, 0.0)
    mu = jnp.sum(ym, axis=0, keepdims=True) * (1.0 / _N)
    d2 = jnp.where(mask, y - mu, 0.0)
    var = jnp.sum(d2 * d2, axis=0, keepdims=True) * (1.0 / _N)
    h = (y - mu) * lax.rsqrt(var + 1e-5) * g_ref[...] + be_ref[...]
    h = jnp.maximum(h, 0.0)
    return jnp.where(mask, h, 0.0), dinv


def _tc_layer_body(p_ref, degp_ref, b_ref, g_ref, be_ref, w_ref, o_ref):
    h, dinv = _bn_relu(p_ref, degp_ref, b_ref, g_ref, be_ref)
    o_ref[...] = jnp.dot(h * dinv, w_ref[...],
                         preferred_element_type=jnp.float32)


_tc_layer = pl.pallas_call(
    _tc_layer_body,
    out_shape=jax.ShapeDtypeStruct((_NPAD, _D), jnp.float32),
)


def _tc_head_body(p_ref, degp_ref, b_ref, g_ref, be_ref, batch_ref,
                  fw1_ref, fb1_ref, fw2_ref, fb2_ref, o_ref):
    h, _ = _bn_relu(p_ref, degp_ref, b_ref, g_ref, be_ref)
    gi = lax.broadcasted_iota(jnp.int32, (_G, 1), 0)
    oh = (batch_ref[...] == gi).astype(jnp.float32)
    pooled_sum = jax.lax.dot_general(
        oh, h, (((1,), (0,)), ((), ())),
        preferred_element_type=jnp.float32)
    counts = jnp.sum(oh, axis=1, keepdims=True)
    pooled = pooled_sum / jnp.maximum(counts, 1.0)
    z = jnp.maximum(
        jnp.dot(pooled, fw1_ref[...], preferred_element_type=jnp.float32)
        + fb1_ref[...], 0.0)
    o_ref[...] = jnp.dot(z, fw2_ref[...],
                         preferred_element_type=jnp.float32) + fb2_ref[...]


_tc_head = pl.pallas_call(
    _tc_head_body,
    out_shape=jax.ShapeDtypeStruct((_G, _C), jnp.float32),
)


def kernel(x, edge_index, batch, W1, b1, g1, be1, W2, b2, g2, be2,
           W3, b3, g3, be3, fW1, fb1, fW2, fb2):
    f32 = jnp.float32
    loop = jnp.arange(_N, dtype=edge_index.dtype)
    pad = _EPAD - _EP
    spare = _N + jnp.arange(pad, dtype=edge_index.dtype) % (_NPAD - _N)
    src = jnp.concatenate([edge_index[0], loop, spare])
    dst = jnp.concatenate([edge_index[1], loop, spare])
    src_p = src.reshape(_NC, _NS, _NWIN, _K)
    dst_p = dst.reshape(_NC, _NS, _NWIN, _K)

    zeros_d = jnp.zeros((_NPAD, _D), f32)
    xp = jnp.pad(x, ((0, _NPAD - _N), (0, 0)))
    batch_p = jnp.pad(batch, (0, _NPAD - _N),
                      constant_values=_G).reshape(1, _NPAD)
    row = lambda v: v.reshape(1, -1)

    degh = _deg_kernel(dst_p)
    degh_t = degh.reshape(_NC * _NS, _NPAD).T
    hs1, degt = _tc_scale(xp, W1, degh_t)
    p1 = _msg_kernel(hs1, src_p, dst_p, zeros_d)
    hs2 = _tc_layer(p1, degt, row(b1), row(g1), row(be1), W2)
    p2 = _msg_kernel(hs2, src_p, dst_p, zeros_d)
    hs3 = _tc_layer(p2, degt, row(b2), row(g2), row(be2), W3)
    p3 = _msg_kernel(hs3, src_p, dst_p, zeros_d)
    return _tc_head(p3, degt, row(b3), row(g3), row(be3), batch_p,
                    fW1, row(fb1), fW2, row(fb2))


# speedup vs baseline: 1.2324x; 1.0048x over previous
import dataclasses
import functools

import jax
import jax.numpy as jnp
from jax import lax
from jax.experimental import pallas as pl
from jax.experimental.pallas import tpu as pltpu
from jax.experimental.pallas import tpu_sc as plsc

_N = 10000
_E = 320000
_D = 128
_G = 64
_C = 16

_NC = 2
_NS = 16
_K = 128

_NPAD = 10112
_RPS = _NPAD // _NS

_EP = _E + _N
_NWIN = 82
_PHASES = (48, 34)
_EPAD = _NC * _NS * _NWIN * _K

_mesh = plsc.VectorSubcoreMesh(core_axis_name="c", subcore_axis_name="s")


@functools.partial(
    pl.kernel,
    out_type=jax.ShapeDtypeStruct((_NC, _NS, _NPAD), jnp.float32),
    mesh=_mesh,
    scratch_types=[
        pltpu.VMEM((_NWIN, _K), jnp.int32),
        pltpu.VMEM((_NPAD,), jnp.float32),
    ],
    compiler_params=dataclasses.replace(
        pltpu.CompilerParams(), needs_layout_passes=False),
)
def _deg_kernel(dst_hbm, out_hbm, dst_v, hist_v):
    c = lax.axis_index("c")
    s = lax.axis_index("s")
    pltpu.sync_copy(dst_hbm.at[c, s], dst_v)

    @pl.loop(0, _NPAD // 16)
    def _(i):
        hist_v[pl.ds(i * 16, 16)] = jnp.zeros((16,), jnp.float32)

    @pl.loop(0, _NWIN)
    def _(j):
        for i in range(_K // 16):
            vec = dst_v[j, pl.ds(i * 16, 16)]
            cnt, lastm = plsc.scan_count(vec)
            old = plsc.load_gather(hist_v, [vec])
            plsc.store_scatter(hist_v, [vec],
                               old + cnt.astype(jnp.float32), mask=lastm)

    pltpu.sync_copy(hist_v, out_hbm.at[c, s])


@functools.partial(
    pl.kernel,
    out_type=jax.ShapeDtypeStruct((_NC, _NPAD, _D), jnp.float32),
    mesh=_mesh,
    scratch_types=[
        pltpu.VMEM((_PHASES[0], _K), jnp.int32),
        pltpu.VMEM((_PHASES[0], _K), jnp.int32),
        pltpu.VMEM((_K, _D), jnp.float32),
        pltpu.VMEM((_K, _D), jnp.float32),
        pltpu.VMEM_SHARED((_NPAD, _D), jnp.float32),
        pltpu.SemaphoreType.DMA,
        pltpu.SemaphoreType.DMA,
        pltpu.SemaphoreType.DMA,
        pltpu.SemaphoreType.DMA,
    ],
)
def _msg_kernel(hs_hbm, src_hbm, dst_hbm, zeros_hbm, out_hbm,
                src_v, dst_v, b0, b1, acc_sh, sg0, sg1, ss0, ss1):
    c = lax.axis_index("c")
    s = lax.axis_index("s")
    rows = pl.ds(s * _RPS, _RPS)
    zc = pltpu.async_copy(zeros_hbm.at[rows], acc_sh.at[rows], ss0)

    off = 0
    for ph, n in enumerate(_PHASES):
        pltpu.sync_copy(src_hbm.at[c, s, pl.ds(off, n)], src_v.at[pl.ds(0, n)])
        pltpu.sync_copy(dst_hbm.at[c, s, pl.ds(off, n)], dst_v.at[pl.ds(0, n)])
        off += n
        if ph == 0:
            zc.wait()
            plsc.subcore_barrier()

        @pl.loop(0, n, step=2)
        def _(j):
            @pl.when(j > 0)
            def _():
                pltpu.make_async_copy(b0, acc_sh.at[dst_v.at[j]], ss0).wait()
            pltpu.sync_copy(hs_hbm.at[src_v.at[j]], b0)
            pltpu.async_copy(b0, acc_sh.at[dst_v.at[j]], ss0, add=True)

            @pl.when(j > 0)
            def _():
                pltpu.make_async_copy(b1, acc_sh.at[dst_v.at[j]], ss1).wait()
            pltpu.sync_copy(hs_hbm.at[src_v.at[j + 1]], b1)
            pltpu.async_copy(b1, acc_sh.at[dst_v.at[j + 1]], ss1, add=True)

        pltpu.make_async_copy(b0, acc_sh.at[dst_v.at[0]], ss0).wait()
        pltpu.make_async_copy(b1, acc_sh.at[dst_v.at[1]], ss1).wait()

    plsc.subcore_barrier()
    pltpu.sync_copy(acc_sh.at[rows], out_hbm.at[c, rows])


def _dinv_from_degp(degt):
    deg = degt[:, 0:1] + degt[:, 1:2]
    return lax.rsqrt(jnp.maximum(deg, 1.0))


def _tc_scale_body(x_ref, w_ref, degh_ref, o_ref, degt_ref):
    deg = jnp.sum(degh_ref[...], axis=1, keepdims=True)
    degt = jnp.concatenate([deg, jnp.zeros_like(deg)], axis=1)
    degt_ref[...] = degt
    hm = jnp.dot(x_ref[...], w_ref[...], preferred_element_type=jnp.float32)
    o_ref[...] = hm * _dinv_from_degp(degt)


_tc_scale = pl.pallas_call(
    _tc_scale_body,
    out_shape=(jax.ShapeDtypeStruct((_NPAD, _D), jnp.float32),
               jax.ShapeDtypeStruct((_NPAD, 2), jnp.float32)),
)


def _bn_relu(p_ref, degp_ref, b_ref, g_ref, be_ref):
    dinv = _dinv_from_degp(degp_ref[...])
    y = (p_ref[0] + p_ref[1]) * dinv + b_ref[...]
    mask = lax.broadcasted_iota(jnp.int32, (_NPAD, 1), 0) < _N
    ym = jnp.where(mask, y, 0.0)
    mu = jnp.sum(ym, axis=0, keepdims=True) * (1.0 / _N)
    d2 = jnp.where(mask, y - mu, 0.0)
    var = jnp.sum(d2 * d2, axis=0, keepdims=True) * (1.0 / _N)
    h = (y - mu) * lax.rsqrt(var + 1e-5) * g_ref[...] + be_ref[...]
    h = jnp.maximum(h, 0.0)
    return jnp.where(mask, h, 0.0), dinv


def _tc_layer_body(p_ref, degp_ref, b_ref, g_ref, be_ref, w_ref, o_ref):
    h, dinv = _bn_relu(p_ref, degp_ref, b_ref, g_ref, be_ref)
    o_ref[...] = jnp.dot(h * dinv, w_ref[...],
                         preferred_element_type=jnp.float32)


_tc_layer = pl.pallas_call(
    _tc_layer_body,
    out_shape=jax.ShapeDtypeStruct((_NPAD, _D), jnp.float32),
)


def _tc_head_body(p_ref, degp_ref, b_ref, g_ref, be_ref, batch_ref,
                  fw1_ref, fb1_ref, fw2_ref, fb2_ref, o_ref):
    h, _ = _bn_relu(p_ref, degp_ref, b_ref, g_ref, be_ref)
    gi = lax.broadcasted_iota(jnp.int32, (_G, 1), 0)
    oh = (batch_ref[...] == gi).astype(jnp.float32)
    pooled_sum = jax.lax.dot_general(
        oh, h, (((1,), (0,)), ((), ())),
        preferred_element_type=jnp.float32)
    counts = jnp.sum(oh, axis=1, keepdims=True)
    pooled = pooled_sum / jnp.maximum(counts, 1.0)
    z = jnp.maximum(
        jnp.dot(pooled, fw1_ref[...], preferred_element_type=jnp.float32)
        + fb1_ref[...], 0.0)
    o_ref[...] = jnp.dot(z, fw2_ref[...],
                         preferred_element_type=jnp.float32) + fb2_ref[...]


_tc_head = pl.pallas_call(
    _tc_head_body,
    out_shape=jax.ShapeDtypeStruct((_G, _C), jnp.float32),
)


def kernel(x, edge_index, batch, W1, b1, g1, be1, W2, b2, g2, be2,
           W3, b3, g3, be3, fW1, fb1, fW2, fb2):
    f32 = jnp.float32
    loop = jnp.arange(_N, dtype=edge_index.dtype)
    pad = _EPAD - _EP
    spare = _N + jnp.arange(pad, dtype=edge_index.dtype) % (_NPAD - _N)
    src = jnp.concatenate([edge_index[0], loop, spare])
    dst = jnp.concatenate([edge_index[1], loop, spare])
    src_p = src.reshape(_NC, _NS, _NWIN, _K)
    dst_p = dst.reshape(_NC, _NS, _NWIN, _K)

    zeros_d = jnp.zeros((_NPAD, _D), f32)
    xp = jnp.pad(x, ((0, _NPAD - _N), (0, 0)))
    batch_p = jnp.pad(batch, (0, _NPAD - _N),
                      constant_values=_G).reshape(1, _NPAD)
    row = lambda v: v.reshape(1, -1)

    degh = _deg_kernel(dst_p)
    degh_t = degh.reshape(_NC * _NS, _NPAD).T
    hs1, degt = _tc_scale(xp, W1, degh_t)
    p1 = _msg_kernel(hs1, src_p, dst_p, zeros_d)
    hs2 = _tc_layer(p1, degt, row(b1), row(g1), row(be1), W2)
    p2 = _msg_kernel(hs2, src_p, dst_p, zeros_d)
    hs3 = _tc_layer(p2, degt, row(b2), row(g2), row(be2), W3)
    p3 = _msg_kernel(hs3, src_p, dst_p, zeros_d)
    return _tc_head(p3, degt, row(b3), row(g3), row(be3), batch_p,
                    fW1, row(fb1), fW2, row(fb2))
